# baseline (device time: 90813 ns/iter reference)
import jax
import jax.numpy as jnp
from jax import lax
from jax.experimental import pallas as pl
from jax.experimental.pallas import tpu as pltpu

N_DEV = 8
PART_ROWS = (192, 160, 160)
PART_OFF = (0, 192, 352)
DIM_MASKS = (1, 3, 4)
PART_DIMS = tuple(tuple(DIM_MASKS[(p + r) % 3] for r in range(3))
                  for p in range(3))


def _origin_mask(p, j):
    m = 0
    for r in range(3):
        if j & (1 << r):
            m ^= PART_DIMS[p][r]
    return m


def kernel(x, w_mat, scale_x, scale_w):
    m_per, k = x.shape
    _, n_per = w_mat.shape

    def body(x_ref, w_ref, sx_ref, sw_ref, out_ref,
             w8_ref, b0, b1, b2, s0, r0, s1, r1, s2, r2):
        bufs = (b0, b1, b2)
        ssems = (s0, s1, s2)
        rsems = (r0, r1, r2)

        my = lax.axis_index("i")

        barrier = pltpu.get_barrier_semaphore()
        for m in DIM_MASKS:
            pl.semaphore_signal(barrier, inc=1,
                                device_id=(jnp.bitwise_xor(my, m),),
                                device_id_type=pl.DeviceIdType.MESH)
        pl.semaphore_wait(barrier, 3)

        scale = sx_ref[0] * sw_ref[0]

        def rdma(p, src_slot, dst_slot, dim_mask):
            return pltpu.make_async_remote_copy(
                src_ref=bufs[p].at[src_slot],
                dst_ref=bufs[p].at[dst_slot],
                send_sem=ssems[p].at[dst_slot - 1],
                recv_sem=rsems[p].at[dst_slot - 1],
                device_id=(jnp.bitwise_xor(my, dim_mask),),
                device_id_type=pl.DeviceIdType.MESH,
            )

        desc = {}
        for p in range(3):
            d0m, d1m, d2m = PART_DIMS[p]
            desc[(p, 1)] = rdma(p, 0, 1, d0m)
            desc[(p, 2)] = rdma(p, 0, 2, d1m)
            desc[(p, 3)] = rdma(p, 1, 3, d1m)
            for j in range(4):
                desc[(p, 4 + j)] = rdma(p, j, 4 + j, d2m)

        def gemm(p, j):
            origin = jnp.bitwise_xor(my, _origin_mask(p, j))
            out_ref[pl.ds(origin * m_per + PART_OFF[p], PART_ROWS[p]), :] = (
                jnp.dot(bufs[p][j], w8_ref[...],
                        preferred_element_type=jnp.float32) * scale
            )

        for p in range(3):
            lo, rows = PART_OFF[p], PART_ROWS[p]
            bufs[p][0] = x_ref[lo:lo + rows, :].astype(jnp.float8_e4m3fn)
            desc[(p, 1)].start()
        for p in range(3):
            desc[(p, 2)].start()
        for p in range(3):
            desc[(p, 4)].start()

        w8_ref[...] = w_ref[...].astype(jnp.float8_e4m3fn)
        for p in range(3):
            gemm(p, 0)

        for p in range(3):
            desc[(p, 1)].wait_recv()
            desc[(p, 3)].start()
            desc[(p, 5)].start()
        for p in range(3):
            gemm(p, 1)

        for p in range(3):
            desc[(p, 2)].wait_recv()
            desc[(p, 6)].start()
        for p in range(3):
            gemm(p, 2)
        for p in range(3):
            desc[(p, 3)].wait_recv()
            desc[(p, 7)].start()
        for p in range(3):
            gemm(p, 3)

        for j in range(4, 8):
            for p in range(3):
                desc[(p, j)].wait_recv()
                gemm(p, j)

        for p in range(3):
            for j in range(1, 8):
                desc[(p, j)].wait_send()

    return pl.pallas_call(
        body,
        out_shape=jax.ShapeDtypeStruct((N_DEV * m_per, n_per), jnp.float32),
        in_specs=[
            pl.BlockSpec(memory_space=pltpu.VMEM),
            pl.BlockSpec(memory_space=pltpu.VMEM),
            pl.BlockSpec(memory_space=pltpu.SMEM),
            pl.BlockSpec(memory_space=pltpu.SMEM),
        ],
        out_specs=pl.BlockSpec(memory_space=pltpu.VMEM),
        scratch_shapes=[
            pltpu.VMEM((k, n_per), jnp.float8_e4m3fn),
            pltpu.VMEM((N_DEV, PART_ROWS[0], k), jnp.float8_e4m3fn),
            pltpu.VMEM((N_DEV, PART_ROWS[1], k), jnp.float8_e4m3fn),
            pltpu.VMEM((N_DEV, PART_ROWS[2], k), jnp.float8_e4m3fn),
            pltpu.SemaphoreType.DMA((7,)), pltpu.SemaphoreType.DMA((7,)),
            pltpu.SemaphoreType.DMA((7,)), pltpu.SemaphoreType.DMA((7,)),
            pltpu.SemaphoreType.DMA((7,)), pltpu.SemaphoreType.DMA((7,)),
        ],
        compiler_params=pltpu.CompilerParams(
            collective_id=0,
            vmem_limit_bytes=100 * 1024 * 1024,
        ),
    )(x, w_mat, scale_x, scale_w)


# device time: 90660 ns/iter; 1.0017x vs baseline; 1.0017x over previous
import jax
import jax.numpy as jnp
from jax import lax
from jax.experimental import pallas as pl
from jax.experimental.pallas import tpu as pltpu

N_DEV = 8
PART_ROWS = (192, 160, 160)
PART_OFF = (0, 192, 352)
DIM_MASKS = (1, 3, 4)
PART_DIMS = tuple(tuple(DIM_MASKS[(p + r) % 3] for r in range(3))
                  for p in range(3))


def _origin_mask(p, j):
    m = 0
    for r in range(3):
        if j & (1 << r):
            m ^= PART_DIMS[p][r]
    return m


def kernel(x, w_mat, scale_x, scale_w):
    m_per, k = x.shape
    _, n_per = w_mat.shape

    def body(x_ref, w_ref, sx_ref, sw_ref, out_ref,
             w8_ref, b0, b1, b2, s0, r0, s1, r1, s2, r2, ready_sems):
        bufs = (b0, b1, b2)
        ssems = (s0, s1, s2)
        rsems = (r0, r1, r2)

        my = lax.axis_index("i")

        barrier = pltpu.get_barrier_semaphore()
        pl.semaphore_signal(barrier, inc=1)
        pl.semaphore_wait(barrier, 1)
        for d, m in enumerate(DIM_MASKS):
            pl.semaphore_signal(ready_sems.at[d], inc=1,
                                device_id=(jnp.bitwise_xor(my, m),),
                                device_id_type=pl.DeviceIdType.MESH)

        scale = sx_ref[0] * sw_ref[0]

        def rdma(p, src_slot, dst_slot, dim_mask):
            return pltpu.make_async_remote_copy(
                src_ref=bufs[p].at[src_slot],
                dst_ref=bufs[p].at[dst_slot],
                send_sem=ssems[p].at[dst_slot - 1],
                recv_sem=rsems[p].at[dst_slot - 1],
                device_id=(jnp.bitwise_xor(my, dim_mask),),
                device_id_type=pl.DeviceIdType.MESH,
            )

        desc = {}
        for p in range(3):
            d0m, d1m, d2m = PART_DIMS[p]
            desc[(p, 1)] = rdma(p, 0, 1, d0m)
            desc[(p, 2)] = rdma(p, 0, 2, d1m)
            desc[(p, 3)] = rdma(p, 1, 3, d1m)
            for j in range(4):
                desc[(p, 4 + j)] = rdma(p, j, 4 + j, d2m)

        def gemm(p, j):
            origin = jnp.bitwise_xor(my, _origin_mask(p, j))
            out_ref[pl.ds(origin * m_per + PART_OFF[p], PART_ROWS[p]), :] = (
                jnp.dot(bufs[p][j], w8_ref[...],
                        preferred_element_type=jnp.float32) * scale
            )

        for p in range(3):
            lo, rows = PART_OFF[p], PART_ROWS[p]
            bufs[p][0] = x_ref[lo:lo + rows, :].astype(jnp.float8_e4m3fn)
            pl.semaphore_wait(ready_sems.at[p], 1)
            desc[(p, 1)].start()
        for p in range(3):
            desc[(p, 2)].start()
        for p in range(3):
            desc[(p, 4)].start()

        w8_ref[...] = w_ref[...].astype(jnp.float8_e4m3fn)
        for p in range(3):
            gemm(p, 0)

        for p in range(3):
            desc[(p, 1)].wait_recv()
            desc[(p, 3)].start()
            desc[(p, 5)].start()
        for p in range(3):
            gemm(p, 1)

        for p in range(3):
            desc[(p, 2)].wait_recv()
            desc[(p, 6)].start()
        for p in range(3):
            gemm(p, 2)
        for p in range(3):
            desc[(p, 3)].wait_recv()
            desc[(p, 7)].start()
        for p in range(3):
            gemm(p, 3)

        for j in range(4, 8):
            for p in range(3):
                desc[(p, j)].wait_recv()
                gemm(p, j)

        for p in range(3):
            for j in range(1, 8):
                desc[(p, j)].wait_send()

    return pl.pallas_call(
        body,
        out_shape=jax.ShapeDtypeStruct((N_DEV * m_per, n_per), jnp.float32),
        in_specs=[
            pl.BlockSpec(memory_space=pltpu.VMEM),
            pl.BlockSpec(memory_space=pltpu.VMEM),
            pl.BlockSpec(memory_space=pltpu.SMEM),
            pl.BlockSpec(memory_space=pltpu.SMEM),
        ],
        out_specs=pl.BlockSpec(memory_space=pltpu.VMEM),
        scratch_shapes=[
            pltpu.VMEM((k, n_per), jnp.float8_e4m3fn),
            pltpu.VMEM((N_DEV, PART_ROWS[0], k), jnp.float8_e4m3fn),
            pltpu.VMEM((N_DEV, PART_ROWS[1], k), jnp.float8_e4m3fn),
            pltpu.VMEM((N_DEV, PART_ROWS[2], k), jnp.float8_e4m3fn),
            pltpu.SemaphoreType.DMA((7,)), pltpu.SemaphoreType.DMA((7,)),
            pltpu.SemaphoreType.DMA((7,)), pltpu.SemaphoreType.DMA((7,)),
            pltpu.SemaphoreType.DMA((7,)), pltpu.SemaphoreType.DMA((7,)),
            pltpu.SemaphoreType.REGULAR((3,)),
        ],
        compiler_params=pltpu.CompilerParams(
            collective_id=0,
            vmem_limit_bytes=100 * 1024 * 1024,
        ),
    )(x, w_mat, scale_x, scale_w)


# device time: 89147 ns/iter; 1.0187x vs baseline; 1.0170x over previous
import jax
import jax.numpy as jnp
from jax import lax
from jax.experimental import pallas as pl
from jax.experimental.pallas import tpu as pltpu

N_DEV = 8
PART_ROWS = (176, 168, 168)
PART_OFF = (0, 176, 344)
DIM_MASKS = (1, 3, 4)
PART_DIMS = tuple(tuple(DIM_MASKS[(p + r) % 3] for r in range(3))
                  for p in range(3))


def _origin_mask(p, j):
    m = 0
    for r in range(3):
        if j & (1 << r):
            m ^= PART_DIMS[p][r]
    return m


def kernel(x, w_mat, scale_x, scale_w):
    m_per, k = x.shape
    _, n_per = w_mat.shape

    def body(x_ref, w_ref, sx_ref, sw_ref, out_ref,
             w8_ref, b0, b1, b2, s0, r0, s1, r1, s2, r2, ready_sems):
        bufs = (b0, b1, b2)
        ssems = (s0, s1, s2)
        rsems = (r0, r1, r2)

        my = lax.axis_index("i")

        barrier = pltpu.get_barrier_semaphore()
        pl.semaphore_signal(barrier, inc=1)
        pl.semaphore_wait(barrier, 1)
        for d, m in enumerate(DIM_MASKS):
            pl.semaphore_signal(ready_sems.at[d], inc=1,
                                device_id=(jnp.bitwise_xor(my, m),),
                                device_id_type=pl.DeviceIdType.MESH)

        scale = sx_ref[0] * sw_ref[0]

        def rdma(p, src_slot, dst_slot, dim_mask):
            return pltpu.make_async_remote_copy(
                src_ref=bufs[p].at[src_slot],
                dst_ref=bufs[p].at[dst_slot],
                send_sem=ssems[p].at[dst_slot - 1],
                recv_sem=rsems[p].at[dst_slot - 1],
                device_id=(jnp.bitwise_xor(my, dim_mask),),
                device_id_type=pl.DeviceIdType.MESH,
            )

        desc = {}
        for p in range(3):
            d0m, d1m, d2m = PART_DIMS[p]
            desc[(p, 1)] = rdma(p, 0, 1, d0m)
            desc[(p, 2)] = rdma(p, 0, 2, d1m)
            desc[(p, 3)] = rdma(p, 1, 3, d1m)
            for j in range(4):
                desc[(p, 4 + j)] = rdma(p, j, 4 + j, d2m)

        def gemm(p, j):
            origin = jnp.bitwise_xor(my, _origin_mask(p, j))
            out_ref[pl.ds(origin * m_per + PART_OFF[p], PART_ROWS[p]), :] = (
                jnp.dot(bufs[p][j], w8_ref[...],
                        preferred_element_type=jnp.float32) * scale
            )

        for p in range(3):
            lo, rows = PART_OFF[p], PART_ROWS[p]
            bufs[p][0] = x_ref[lo:lo + rows, :].astype(jnp.float8_e4m3fn)
            pl.semaphore_wait(ready_sems.at[p], 1)
            desc[(p, 1)].start()
        for p in range(3):
            desc[(p, 2)].start()
        for p in range(3):
            desc[(p, 4)].start()

        w8_ref[...] = w_ref[...].astype(jnp.float8_e4m3fn)
        for p in range(3):
            gemm(p, 0)

        for p in range(3):
            desc[(p, 1)].wait_recv()
            desc[(p, 3)].start()
            desc[(p, 5)].start()
        for p in range(3):
            gemm(p, 1)

        for p in range(3):
            desc[(p, 2)].wait_recv()
            desc[(p, 6)].start()
        for p in range(3):
            gemm(p, 2)
        for p in range(3):
            desc[(p, 3)].wait_recv()
            desc[(p, 7)].start()
        for p in range(3):
            gemm(p, 3)

        for j in range(4, 8):
            for p in range(3):
                desc[(p, j)].wait_recv()
                gemm(p, j)

        for p in range(3):
            for j in range(1, 8):
                desc[(p, j)].wait_send()

    return pl.pallas_call(
        body,
        out_shape=jax.ShapeDtypeStruct((N_DEV * m_per, n_per), jnp.float32),
        in_specs=[
            pl.BlockSpec(memory_space=pltpu.VMEM),
            pl.BlockSpec(memory_space=pltpu.VMEM),
            pl.BlockSpec(memory_space=pltpu.SMEM),
            pl.BlockSpec(memory_space=pltpu.SMEM),
        ],
        out_specs=pl.BlockSpec(memory_space=pltpu.VMEM),
        scratch_shapes=[
            pltpu.VMEM((k, n_per), jnp.float8_e4m3fn),
            pltpu.VMEM((N_DEV, PART_ROWS[0], k), jnp.float8_e4m3fn),
            pltpu.VMEM((N_DEV, PART_ROWS[1], k), jnp.float8_e4m3fn),
            pltpu.VMEM((N_DEV, PART_ROWS[2], k), jnp.float8_e4m3fn),
            pltpu.SemaphoreType.DMA((7,)), pltpu.SemaphoreType.DMA((7,)),
            pltpu.SemaphoreType.DMA((7,)), pltpu.SemaphoreType.DMA((7,)),
            pltpu.SemaphoreType.DMA((7,)), pltpu.SemaphoreType.DMA((7,)),
            pltpu.SemaphoreType.REGULAR((3,)),
        ],
        compiler_params=pltpu.CompilerParams(
            collective_id=0,
            vmem_limit_bytes=100 * 1024 * 1024,
        ),
    )(x, w_mat, scale_x, scale_w)
